# baseline (speedup 1.0000x reference)
import functools

import jax
import jax.numpy as jnp
from jax import lax
from jax.experimental import pallas as pl
from jax.experimental.pallas import tpu as pltpu
from jax.experimental.pallas import tpu_sc as plsc

VOCAB = 1000000
DIM = 64
B = 16384
L = 50
NC = 2
NS = 16
NW = NC * NS
LANES = 16
BCHUNK = B // NW


def _make_kernel():
    n_total = B * L
    mesh = plsc.VectorSubcoreMesh(
        core_axis_name="c", subcore_axis_name="s",
        num_cores=NC, num_subcores=NS,
    )

    @functools.partial(
        pl.kernel,
        mesh=mesh,
        compiler_params=pltpu.CompilerParams(use_tc_tiling_on_sc=False),
        out_type=jax.ShapeDtypeStruct((n_total, DIM), jnp.float32),
        scratch_types=[
            pltpu.VMEM((BCHUNK,), jnp.int32),
            pltpu.VMEM((BCHUNK, DIM), jnp.float32),
            pltpu.VMEM((BCHUNK,), jnp.float32),
            pltpu.VMEM((BCHUNK,), jnp.int32),
            pltpu.SemaphoreType.DMA,
            pltpu.SemaphoreType.DMA,
            pltpu.SemaphoreType.DMA,
        ],
    )
    def k(words_hbm, table_hbm, mask_hbm, out_hbm, idx_v, rows_v, maskv_v,
          ridx_v, sem_r, sem_m, sem_w):
        wid = lax.axis_index("s") * NC + lax.axis_index("c")
        b0 = wid * BCHUNK

        lane = lax.iota(jnp.int32, LANES)
        for i in range(BCHUNK // LANES):
            ridx_v[pl.ds(i * LANES, LANES)] = (b0 + i * LANES + lane) * L

        def do_chunk(l, carry):
            src = l * B + b0
            pltpu.sync_copy(words_hbm.at[pl.ds(src, BCHUNK)], idx_v)
            cp_r = pltpu.async_copy(table_hbm.at[idx_v], rows_v, sem_r)
            cp_m = pltpu.async_copy(mask_hbm.at[idx_v], maskv_v, sem_m)
            cp_r.wait()
            cp_m.wait()

            def rowgrp(g16, c):
                mvec = maskv_v[pl.ds(g16 * LANES, LANES)]
                for r in range(LANES):
                    i = g16 * LANES + r
                    m = mvec[r]
                    for j in range(DIM // LANES):
                        sl = pl.ds(j * LANES, LANES)
                        rows_v[i, sl] = rows_v[i, sl] * m
                return c

            lax.fori_loop(0, BCHUNK // LANES, rowgrp, 0)

            pltpu.async_copy(rows_v, out_hbm.at[ridx_v], sem_w).wait()
            for i in range(BCHUNK // LANES):
                sl = pl.ds(i * LANES, LANES)
                ridx_v[sl] = ridx_v[sl] + 1
            return carry

        lax.fori_loop(0, L, do_chunk, 0)

    return k


_KERNEL = _make_kernel()


@jax.jit
def kernel(words, raw_weight, mask):
    flat_words = words.T.reshape(-1).astype(jnp.int32)
    flat_mask = mask.reshape(-1)
    out = _KERNEL(flat_words, raw_weight, flat_mask)
    return out.reshape(B, L, DIM)
